# Initial kernel scaffold; baseline (speedup 1.0000x reference)
import jax
import jax.numpy as jnp
from jax.experimental import pallas as pl


def kernel(indices, table):
    raise NotImplementedError("write your pallas kernel here")


# speedup vs baseline: 3.2507x; 3.2507x over previous
import jax
import jax.numpy as jnp
from jax.experimental import pallas as pl
from jax.experimental.pallas import tpu as pltpu
from jax.experimental.pallas import tpu_sc as plsc

VOCAB = 50
POS_DIM = 64
WINDOW = 128


def _sc_gather(pair_table, idx_flat, n_pairs):
    mesh = plsc.VectorSubcoreMesh(core_axis_name="core", subcore_axis_name="subcore")

    @pl.kernel(
        out_type=jax.ShapeDtypeStruct((n_pairs, 2 * POS_DIM), pair_table.dtype),
        mesh=mesh,
    )
    def gather_kernel(table_hbm, idx_hbm, out_hbm):
        def body(idx_vmem, out_vmem):
            pltpu.sync_copy(table_hbm.at[idx_vmem.at[0]], out_vmem)

        pltpu.emit_pipeline(
            body,
            grid=(n_pairs // WINDOW,),
            in_specs=[pl.BlockSpec((1, WINDOW), index_map=lambda i: (0, i))],
            out_specs=[
                pl.BlockSpec((WINDOW, 2 * POS_DIM), index_map=lambda i: (i, 0))
            ],
            core_axis_name=("core", "subcore"),
            dimension_semantics=(pltpu.PARALLEL,),
        )(idx_hbm, out_hbm)

    return gather_kernel(pair_table, idx_flat)


def kernel(indices, table):
    batch, seq_len = indices.shape
    total = batch * seq_len
    n_pairs = total // 2

    pair_table = jnp.concatenate(
        [
            jnp.broadcast_to(table[:, None, :], (VOCAB, VOCAB, POS_DIM)),
            jnp.broadcast_to(table[None, :, :], (VOCAB, VOCAB, POS_DIM)),
        ],
        axis=-1,
    ).reshape(VOCAB * VOCAB, 2 * POS_DIM)

    idx = indices.reshape(n_pairs, 2).astype(jnp.int32)
    pair_idx = (idx[:, 0] * VOCAB + idx[:, 1]).reshape(1, n_pairs)

    out = _sc_gather(pair_table, pair_idx, n_pairs)
    return out.reshape(batch, seq_len, POS_DIM)
